# baseline (speedup 1.0000x reference)
import functools

import jax
import jax.numpy as jnp
from jax import lax
from jax.experimental import pallas as pl
from jax.experimental.pallas import tpu as pltpu
from jax.experimental.pallas import tpu_sc as plsc

N_NODES = 10000
N_EDGES = 320000
FEAT = 128
MAXD = 10

NC = 2
NS = 16
NW = NC * NS
EPW = N_EDGES // NW
ECH = 96
TAIL = EPW - (EPW // ECH) * ECH
ROWS_A = 624
ROWS_TAIL = N_NODES - ROWS_A * NS


def _sc_segment_sum(table, src, dst, zeros, ech, NB, with_deg):
    nchunk = EPW // ech
    assert nchunk % NB == 0
    ngrp = nchunk // NB
    tail = EPW - nchunk * ech
    mesh = plsc.VectorSubcoreMesh(core_axis_name="c", subcore_axis_name="s")

    out_type = [jax.ShapeDtypeStruct((NC, N_NODES, FEAT), jnp.float32)]
    deg_scratch = []
    if with_deg:
        out_type.append(
            jax.ShapeDtypeStruct((N_NODES // BLK, NW, BLK), jnp.float32))
        deg_scratch.append(pltpu.VMEM((N_NODES // BLK, BLK), jnp.float32))

    @functools.partial(
        pl.kernel,
        out_type=out_type,
        mesh=mesh,
        scratch_types=[
            [pltpu.VMEM((ech,), jnp.int32) for _ in range(NB)],
            [pltpu.VMEM((ech,), jnp.int32) for _ in range(NB)],
            [pltpu.VMEM((ech, FEAT), jnp.float32) for _ in range(NB)],
            pltpu.VMEM((TAIL,), jnp.int32),
            pltpu.VMEM((TAIL,), jnp.int32),
            pltpu.VMEM_SHARED((N_NODES, FEAT), jnp.float32),
            [pltpu.SemaphoreType.DMA for _ in range(NB)],
            [pltpu.SemaphoreType.DMA for _ in range(NB)],
            [pltpu.SemaphoreType.DMA for _ in range(NB)],
            [pltpu.SemaphoreType.DMA for _ in range(NB)],
        ] + deg_scratch,
        compiler_params=pltpu.CompilerParams(needs_layout_passes=False),
    )
    def k(table_hbm, src_hbm, dst_hbm, zeros_hbm, *out_scratch):
        if with_deg:
            (out_hbm, outd_hbm, srcbuf, dstbuf, rows, tsrc, tdst, acc_sh,
             gsem, rsem, dsem, ssem, hist_v) = out_scratch
        else:
            (out_hbm, srcbuf, dstbuf, rows, tsrc, tdst, acc_sh,
             gsem, rsem, dsem, ssem) = out_scratch
            hist_v = None
        ECH = ech
        NCHUNK = nchunk
        NGRP = ngrp
        cid = lax.axis_index("c")
        sid = lax.axis_index("s")
        wid = sid * NC + cid

        if with_deg:
            zeros16 = jnp.zeros((16,), jnp.float32)

            def zstep(i, _):
                hist_v[i // (BLK // 16), pl.ds((i % (BLK // 16)) * 16, 16)] = (
                    zeros16)
                return ()

            lax.fori_loop(0, N_NODES // 16, zstep, ())
            ones16 = jnp.ones((16,), jnp.float32)

        def hist_update(buf, n):
            if with_deg:
                for j in range(n // 16):
                    u = buf[pl.ds(j * 16, 16)]
                    plsc.addupdate_scatter(hist_v, [u // BLK, u % BLK], ones16)

        row0 = sid * ROWS_A
        pltpu.sync_copy(zeros_hbm.at[pl.ds(row0, ROWS_A)],
                        acc_sh.at[pl.ds(row0, ROWS_A)])

        @pl.when(sid == NS - 1)
        def _():
            pltpu.sync_copy(zeros_hbm.at[pl.ds(ROWS_A * NS, ROWS_TAIL)],
                            acc_sh.at[pl.ds(ROWS_A * NS, ROWS_TAIL)])

        plsc.subcore_barrier()

        def group(g, _):
            for b in range(NB):
                @pl.when(g > 0)
                def _():
                    pltpu.make_async_copy(table_hbm.at[pl.ds(0, ECH)], rows[b],
                                          ssem[b]).wait()
                base = wid * EPW + (g * NB + b) * ECH
                pltpu.async_copy(src_hbm.at[pl.ds(base, ECH)], srcbuf[b], rsem[b])
                pltpu.async_copy(dst_hbm.at[pl.ds(base, ECH)], dstbuf[b], dsem[b])
            for b in range(NB):
                pltpu.make_async_copy(src_hbm.at[pl.ds(0, ECH)], srcbuf[b],
                                      rsem[b]).wait()
                pltpu.async_copy(table_hbm.at[srcbuf[b]], rows[b], gsem[b])
            for b in range(NB):
                pltpu.make_async_copy(table_hbm.at[pl.ds(0, ECH)], rows[b],
                                      gsem[b]).wait()
                pltpu.make_async_copy(dst_hbm.at[pl.ds(0, ECH)], dstbuf[b],
                                      dsem[b]).wait()
                pltpu.async_copy(rows[b], acc_sh.at[dstbuf[b]], ssem[b],
                                 add=True)
                hist_update(dstbuf[b], ECH)
            return ()

        lax.fori_loop(0, NGRP, group, ())
        for b in range(NB):
            pltpu.make_async_copy(table_hbm.at[pl.ds(0, ECH)], rows[b],
                                  ssem[b]).wait()
        tbase = wid * EPW + NCHUNK * ECH
        pltpu.sync_copy(src_hbm.at[pl.ds(tbase, TAIL)], tsrc)
        pltpu.sync_copy(dst_hbm.at[pl.ds(tbase, TAIL)], tdst)
        trows = rows[0].at[pl.ds(0, TAIL)]
        pltpu.async_copy(table_hbm.at[tsrc], trows, gsem[0]).wait()
        pltpu.async_copy(trows, acc_sh.at[tdst], ssem[0], add=True).wait()
        hist_update(tdst, TAIL)
        if with_deg:
            pltpu.sync_copy(hist_v, outd_hbm.at[:, wid])
        plsc.subcore_barrier()

        pltpu.sync_copy(acc_sh.at[pl.ds(row0, ROWS_A)],
                        out_hbm.at[cid, pl.ds(row0, ROWS_A)])

        @pl.when(sid == NS - 1)
        def _():
            pltpu.sync_copy(acc_sh.at[pl.ds(ROWS_A * NS, ROWS_TAIL)],
                            out_hbm.at[cid, pl.ds(ROWS_A * NS, ROWS_TAIL)])

    return k(table, src, dst, zeros)


BLK = 2000
NBLK = N_NODES // BLK


def _tc_layer_body(p_ref, x_ref, degp_ref, wcat_ref, bl_ref, wlin_ref,
                   h_ref, pool_ref):
    i = pl.program_id(0)
    hagg = p_ref[0] + p_ref[1]
    degf = jnp.sum(degp_ref[0], axis=0)[:, None]
    degf = jnp.minimum(degf, float(MAXD))
    hx = jnp.concatenate([hagg, x_ref[...]], axis=1)
    hx = hx.astype(jnp.bfloat16)

    acc = jnp.dot(hx, wcat_ref[MAXD], preferred_element_type=jnp.float32)
    acc = acc + bl_ref[MAXD][None, :]

    def low_degree_fix(a):
        for d in range(MAXD):
            r = jnp.dot(hx, wcat_ref[d], preferred_element_type=jnp.float32)
            r = r + bl_ref[d][None, :]
            a = jnp.where(degf == float(d), r, a)
        return a

    acc = lax.cond(jnp.any(degf < float(MAXD)), low_degree_fix, lambda a: a, acc)

    h = jax.nn.sigmoid(acc)
    z = jnp.dot(h, wlin_ref[...], preferred_element_type=jnp.float32)
    e = jnp.exp(z)
    y = e / jnp.sum(e, axis=-1, keepdims=True)

    h_ref[...] = h

    @pl.when(i == 0)
    def _():
        pool_ref[...] = jnp.zeros_like(pool_ref)
    pool_ref[...] += jnp.sum(y, axis=0, keepdims=True)


def _tc_layer(p, x, degp, wcat, bl_l, wlin_l):
    grid = (N_NODES // BLK,)
    return pl.pallas_call(
        _tc_layer_body,
        grid=grid,
        in_specs=[
            pl.BlockSpec((NC, BLK, FEAT), lambda i: (0, i, 0)),
            pl.BlockSpec((BLK, FEAT), lambda i: (i, 0)),
            pl.BlockSpec((1, NW, BLK), lambda i: (i, 0, 0)),
            pl.BlockSpec((MAXD + 1, 2 * FEAT, FEAT), lambda i: (0, 0, 0)),
            pl.BlockSpec((MAXD + 1, FEAT), lambda i: (0, 0)),
            pl.BlockSpec((FEAT, FEAT), lambda i: (0, 0)),
        ],
        out_specs=[
            pl.BlockSpec((BLK, FEAT), lambda i: (i, 0)),
            pl.BlockSpec((1, FEAT), lambda i: (0, 0)),
        ],
        out_shape=[
            jax.ShapeDtypeStruct((N_NODES, FEAT), jnp.float32),
            jax.ShapeDtypeStruct((1, FEAT), jnp.float32),
        ],
    )(p, x, degp, wcat, bl_l, wlin_l)


def kernel(x, edge_index, Wl, bl, Wr, Wlin):
    src = edge_index[0]
    dst = edge_index[1]

    wcat = jnp.concatenate([Wl, Wr], axis=2).astype(jnp.bfloat16)

    zeros = jnp.zeros((N_NODES, FEAT), jnp.float32)
    p, degp = _sc_segment_sum(x, src, dst, zeros, 64, 4, True)
    h, out = _tc_layer(p, x, degp, wcat[0], bl[0], Wlin[0])
    for l in range(1, 3):
        (p,) = _sc_segment_sum(h, src, dst, zeros, 96, 4, False)
        h, pooled = _tc_layer(p, h, degp, wcat[l], bl[l], Wlin[l])
        out = out + pooled
    return out


# speedup vs baseline: 1.0148x; 1.0148x over previous
import functools

import jax
import jax.numpy as jnp
from jax import lax
from jax.experimental import pallas as pl
from jax.experimental.pallas import tpu as pltpu
from jax.experimental.pallas import tpu_sc as plsc

N_NODES = 10000
N_EDGES = 320000
FEAT = 128
MAXD = 10

NC = 2
NS = 16
NW = NC * NS
EPW = N_EDGES // NW
ECH = 96
TAIL = EPW - (EPW // ECH) * ECH
ROWS_A = 624
ROWS_TAIL = N_NODES - ROWS_A * NS


def _sc_segment_sum(table, src, dst, zeros, ech, NB, with_deg):
    nchunk = EPW // ech
    assert nchunk % NB == 0
    ngrp = nchunk // NB
    tail = EPW - nchunk * ech
    mesh = plsc.VectorSubcoreMesh(core_axis_name="c", subcore_axis_name="s")

    out_type = [jax.ShapeDtypeStruct((NC, N_NODES, FEAT), jnp.float32)]
    deg_scratch = []
    if with_deg:
        out_type.append(
            jax.ShapeDtypeStruct((N_NODES // BLK, NW, BLK), jnp.float32))
        deg_scratch.append(pltpu.VMEM((N_NODES // BLK, BLK), jnp.float32))

    @functools.partial(
        pl.kernel,
        out_type=out_type,
        mesh=mesh,
        scratch_types=[
            [pltpu.VMEM((ech,), jnp.int32) for _ in range(NB)],
            [pltpu.VMEM((ech,), jnp.int32) for _ in range(NB)],
            [pltpu.VMEM((ech, FEAT), jnp.float32) for _ in range(NB)],
            pltpu.VMEM((TAIL,), jnp.int32),
            pltpu.VMEM((TAIL,), jnp.int32),
            pltpu.VMEM_SHARED((N_NODES, FEAT), jnp.float32),
            [pltpu.SemaphoreType.DMA for _ in range(NB)],
            [pltpu.SemaphoreType.DMA for _ in range(NB)],
            [pltpu.SemaphoreType.DMA for _ in range(NB)],
            [pltpu.SemaphoreType.DMA for _ in range(NB)],
        ] + deg_scratch,
        compiler_params=pltpu.CompilerParams(needs_layout_passes=False),
    )
    def k(table_hbm, src_hbm, dst_hbm, zeros_hbm, *out_scratch):
        if with_deg:
            (out_hbm, outd_hbm, srcbuf, dstbuf, rows, tsrc, tdst, acc_sh,
             gsem, rsem, dsem, ssem, hist_v) = out_scratch
        else:
            (out_hbm, srcbuf, dstbuf, rows, tsrc, tdst, acc_sh,
             gsem, rsem, dsem, ssem) = out_scratch
            hist_v = None
        ECH = ech
        NCHUNK = nchunk
        NGRP = ngrp
        cid = lax.axis_index("c")
        sid = lax.axis_index("s")
        wid = sid * NC + cid

        if with_deg:
            zeros16 = jnp.zeros((16,), jnp.float32)

            def zstep(i, _):
                hist_v[i // (BLK // 16), pl.ds((i % (BLK // 16)) * 16, 16)] = (
                    zeros16)
                return ()

            lax.fori_loop(0, N_NODES // 16, zstep, ())
            ones16 = jnp.ones((16,), jnp.float32)

        def hist_update(buf, n):
            if with_deg:
                for j in range(n // 16):
                    u = buf[pl.ds(j * 16, 16)]
                    plsc.addupdate_scatter(hist_v, [u // BLK, u % BLK], ones16)

        for b in range(NB):
            pltpu.async_copy(src_hbm.at[pl.ds(wid * EPW + b * ECH, ECH)],
                             srcbuf[b], rsem[b])
            pltpu.async_copy(dst_hbm.at[pl.ds(wid * EPW + b * ECH, ECH)],
                             dstbuf[b], dsem[b])
        for b in range(NB):
            pltpu.make_async_copy(src_hbm.at[pl.ds(0, ECH)], srcbuf[b],
                                  rsem[b]).wait()
            pltpu.async_copy(table_hbm.at[srcbuf[b]], rows[b], gsem[b])

        row0 = sid * ROWS_A
        pltpu.sync_copy(zeros_hbm.at[pl.ds(row0, ROWS_A)],
                        acc_sh.at[pl.ds(row0, ROWS_A)])

        @pl.when(sid == NS - 1)
        def _():
            pltpu.sync_copy(zeros_hbm.at[pl.ds(ROWS_A * NS, ROWS_TAIL)],
                            acc_sh.at[pl.ds(ROWS_A * NS, ROWS_TAIL)])

        plsc.subcore_barrier()

        def group(g, _):
            for b in range(NB):
                @pl.when(g > 0)
                def _():
                    pltpu.make_async_copy(table_hbm.at[pl.ds(0, ECH)], rows[b],
                                          ssem[b]).wait()
                    base = wid * EPW + (g * NB + b) * ECH
                    pltpu.async_copy(src_hbm.at[pl.ds(base, ECH)], srcbuf[b],
                                     rsem[b])
                    pltpu.async_copy(dst_hbm.at[pl.ds(base, ECH)], dstbuf[b],
                                     dsem[b])
            for b in range(NB):
                @pl.when(g > 0)
                def _():
                    pltpu.make_async_copy(src_hbm.at[pl.ds(0, ECH)], srcbuf[b],
                                          rsem[b]).wait()
                    pltpu.async_copy(table_hbm.at[srcbuf[b]], rows[b], gsem[b])
            for b in range(NB):
                pltpu.make_async_copy(table_hbm.at[pl.ds(0, ECH)], rows[b],
                                      gsem[b]).wait()
                pltpu.make_async_copy(dst_hbm.at[pl.ds(0, ECH)], dstbuf[b],
                                      dsem[b]).wait()
                pltpu.async_copy(rows[b], acc_sh.at[dstbuf[b]], ssem[b],
                                 add=True)
                hist_update(dstbuf[b], ECH)
            return ()

        lax.fori_loop(0, NGRP, group, ())
        for b in range(NB):
            pltpu.make_async_copy(table_hbm.at[pl.ds(0, ECH)], rows[b],
                                  ssem[b]).wait()
        tbase = wid * EPW + NCHUNK * ECH
        pltpu.sync_copy(src_hbm.at[pl.ds(tbase, TAIL)], tsrc)
        pltpu.sync_copy(dst_hbm.at[pl.ds(tbase, TAIL)], tdst)
        trows = rows[0].at[pl.ds(0, TAIL)]
        pltpu.async_copy(table_hbm.at[tsrc], trows, gsem[0]).wait()
        pltpu.async_copy(trows, acc_sh.at[tdst], ssem[0], add=True).wait()
        hist_update(tdst, TAIL)
        if with_deg:
            pltpu.sync_copy(hist_v, outd_hbm.at[:, wid])
        plsc.subcore_barrier()

        pltpu.sync_copy(acc_sh.at[pl.ds(row0, ROWS_A)],
                        out_hbm.at[cid, pl.ds(row0, ROWS_A)])

        @pl.when(sid == NS - 1)
        def _():
            pltpu.sync_copy(acc_sh.at[pl.ds(ROWS_A * NS, ROWS_TAIL)],
                            out_hbm.at[cid, pl.ds(ROWS_A * NS, ROWS_TAIL)])

    return k(table, src, dst, zeros)


BLK = 2000
NBLK = N_NODES // BLK


def _tc_layer_body(p_ref, x_ref, degp_ref, wcat_ref, bl_ref, wlin_ref,
                   h_ref, pool_ref):
    i = pl.program_id(0)
    hagg = p_ref[0] + p_ref[1]
    degf = jnp.sum(degp_ref[0], axis=0)[:, None]
    degf = jnp.minimum(degf, float(MAXD))
    hx = jnp.concatenate([hagg, x_ref[...]], axis=1)
    hx = hx.astype(jnp.bfloat16)

    acc = jnp.dot(hx, wcat_ref[MAXD], preferred_element_type=jnp.float32)
    acc = acc + bl_ref[MAXD][None, :]

    def low_degree_fix(a):
        for d in range(MAXD):
            r = jnp.dot(hx, wcat_ref[d], preferred_element_type=jnp.float32)
            r = r + bl_ref[d][None, :]
            a = jnp.where(degf == float(d), r, a)
        return a

    acc = lax.cond(jnp.any(degf < float(MAXD)), low_degree_fix, lambda a: a, acc)

    h = jax.nn.sigmoid(acc)
    z = jnp.dot(h, wlin_ref[...], preferred_element_type=jnp.float32)
    e = jnp.exp(z)
    y = e / jnp.sum(e, axis=-1, keepdims=True)

    h_ref[...] = h

    @pl.when(i == 0)
    def _():
        pool_ref[...] = jnp.zeros_like(pool_ref)
    pool_ref[...] += jnp.sum(y, axis=0, keepdims=True)


def _tc_layer(p, x, degp, wcat, bl_l, wlin_l):
    grid = (N_NODES // BLK,)
    return pl.pallas_call(
        _tc_layer_body,
        grid=grid,
        in_specs=[
            pl.BlockSpec((NC, BLK, FEAT), lambda i: (0, i, 0)),
            pl.BlockSpec((BLK, FEAT), lambda i: (i, 0)),
            pl.BlockSpec((1, NW, BLK), lambda i: (i, 0, 0)),
            pl.BlockSpec((MAXD + 1, 2 * FEAT, FEAT), lambda i: (0, 0, 0)),
            pl.BlockSpec((MAXD + 1, FEAT), lambda i: (0, 0)),
            pl.BlockSpec((FEAT, FEAT), lambda i: (0, 0)),
        ],
        out_specs=[
            pl.BlockSpec((BLK, FEAT), lambda i: (i, 0)),
            pl.BlockSpec((1, FEAT), lambda i: (0, 0)),
        ],
        out_shape=[
            jax.ShapeDtypeStruct((N_NODES, FEAT), jnp.float32),
            jax.ShapeDtypeStruct((1, FEAT), jnp.float32),
        ],
    )(p, x, degp, wcat, bl_l, wlin_l)


def kernel(x, edge_index, Wl, bl, Wr, Wlin):
    src = edge_index[0]
    dst = edge_index[1]

    wcat = jnp.concatenate([Wl, Wr], axis=2).astype(jnp.bfloat16)

    zeros = jnp.zeros((N_NODES, FEAT), jnp.float32)
    p, degp = _sc_segment_sum(x, src, dst, zeros, 64, 4, True)
    h, out = _tc_layer(p, x, degp, wcat[0], bl[0], Wlin[0])
    for l in range(1, 3):
        (p,) = _sc_segment_sum(h, src, dst, zeros, 96, 4, False)
        h, pooled = _tc_layer(p, h, degp, wcat[l], bl[l], Wlin[l])
        out = out + pooled
    return out
